# baseline (speedup 1.0000x reference)
import functools

import jax
import jax.numpy as jnp
from jax import lax
from jax.experimental import pallas as pl
from jax.experimental.pallas import tpu as pltpu
from jax.experimental.pallas import tpu_sc as plsc

N_NODES = 10000
N_EDGES = 320000
D = 128

NC = 2
NS = 16
NW = NC * NS

K = 128
G = 2
NG = 40
EDGES_PER_TILE = G * K * NG
E_PAD = NW * EDGES_PER_TILE
ACC_ROWS = 10240
ROWS_PER_TILE = ACC_ROWS // NS


def _sc_body(feat_hbm, idx_hbm, zacc_hbm, zdeg_hbm, ones_hbm,
             p_hbm, degp_hbm,
             cidx_v, rows_v, ones_v, acc_sh, deg_sh,
             gsem0, gsem1, dsem):
    cid = lax.axis_index("c")
    sid = lax.axis_index("s")
    wid = cid * NS + sid

    r0 = sid * ROWS_PER_TILE
    pltpu.sync_copy(zacc_hbm.at[pl.ds(r0, ROWS_PER_TILE)],
                    acc_sh.at[pl.ds(r0, ROWS_PER_TILE)])
    pltpu.sync_copy(zdeg_hbm.at[pl.ds(r0, ROWS_PER_TILE)],
                    deg_sh.at[pl.ds(r0, ROWS_PER_TILE)])
    pltpu.sync_copy(ones_hbm, ones_v)
    plsc.subcore_barrier()

    def group(g, carry):
        pltpu.sync_copy(idx_hbm.at[wid, g], cidx_v)
        gd0 = pltpu.async_copy(feat_hbm.at[cidx_v.at[0, 0]],
                               rows_v.at[0], gsem0)
        gd1 = pltpu.async_copy(feat_hbm.at[cidx_v.at[0, 1]],
                               rows_v.at[1], gsem1)
        gd0.wait()
        pltpu.sync_copy(rows_v.at[0], acc_sh.at[cidx_v.at[1, 0]], add=True)
        gd1.wait()
        pltpu.sync_copy(rows_v.at[1], acc_sh.at[cidx_v.at[1, 1]], add=True)
        pltpu.async_copy(ones_v, deg_sh.at[cidx_v.at[1, 0]], dsem, add=True)
        pltpu.async_copy(ones_v, deg_sh.at[cidx_v.at[1, 1]], dsem, add=True)
        return carry

    lax.fori_loop(0, NG, group, 0)
    pltpu.make_async_copy(feat_hbm.at[pl.ds(0, NG * G)],
                          rows_v.at[0, pl.ds(0, NG * G)], dsem).wait()
    plsc.subcore_barrier()

    pltpu.sync_copy(acc_sh.at[pl.ds(r0, ROWS_PER_TILE)],
                    p_hbm.at[cid, pl.ds(r0, ROWS_PER_TILE)])
    pltpu.sync_copy(deg_sh.at[pl.ds(r0, ROWS_PER_TILE)],
                    degp_hbm.at[pl.ds(cid * ACC_ROWS + r0, ROWS_PER_TILE)])


_sc_scatter = functools.partial(
    pl.kernel,
    out_type=(jax.ShapeDtypeStruct((NC, ACC_ROWS, D), jnp.float32),
              jax.ShapeDtypeStruct((NC * ACC_ROWS,), jnp.float32)),
    mesh=plsc.VectorSubcoreMesh(core_axis_name="c", subcore_axis_name="s",
                                num_cores=NC, num_subcores=NS),
    scratch_types=[
        pltpu.VMEM((2, G, K), jnp.int32),
        pltpu.VMEM((G, K, D), jnp.float32),
        pltpu.VMEM((K,), jnp.float32),
        pltpu.VMEM_SHARED((ACC_ROWS, D), jnp.float32),
        pltpu.VMEM_SHARED((ACC_ROWS,), jnp.float32),
        pltpu.SemaphoreType.DMA,
        pltpu.SemaphoreType.DMA,
        pltpu.SemaphoreType.DMA,
    ],
)(_sc_body)


def _tc_body(p_ref, deg_ref, feat_ref, w_ref, b_ref, out_ref):
    s = p_ref[0] + p_ref[1]
    d = deg_ref[0] + deg_ref[1]
    mean = s / jnp.maximum(d, 1.0)
    h = jnp.where(d > 0, mean, feat_ref[...])
    y = lax.dot_general(h, w_ref[...], (((1,), (1,)), ((), ())),
                        preferred_element_type=jnp.float32)
    out_ref[...] = jnp.maximum(y + b_ref[...], 0.0)


TC_R = 1280


def _tc_apply(p, degp, featpad, W, b2):
    return pl.pallas_call(
        _tc_body,
        grid=(ACC_ROWS // TC_R,),
        in_specs=[
            pl.BlockSpec((NC, TC_R, D), lambda i: (0, i, 0)),
            pl.BlockSpec((NC, TC_R, 1), lambda i: (0, i, 0)),
            pl.BlockSpec((TC_R, D), lambda i: (i, 0)),
            pl.BlockSpec((D, D), lambda i: (0, 0)),
            pl.BlockSpec((1, D), lambda i: (0, 0)),
        ],
        out_specs=pl.BlockSpec((TC_R, D), lambda i: (i, 0)),
        out_shape=jax.ShapeDtypeStruct((ACC_ROWS, D), jnp.float32),
    )(p, degp, featpad, W, b2)


def kernel(feature, edge_index, W, b):
    pad = E_PAD - N_EDGES
    src = jnp.concatenate([edge_index[0], jnp.zeros((pad,), jnp.int32)])
    dst = jnp.concatenate(
        [edge_index[1], jnp.full((pad,), N_NODES, jnp.int32)])
    src5 = src.reshape(NW, NG, 1, G, K)
    dst5 = dst.reshape(NW, NG, 1, G, K)
    idx5 = jnp.concatenate([src5, dst5], axis=2)
    zacc = jnp.zeros((ACC_ROWS, D), jnp.float32)
    zdeg = jnp.zeros((ACC_ROWS,), jnp.float32)
    ones_k = jnp.ones((K,), jnp.float32)

    p, degp = _sc_scatter(feature, idx5, zacc, zdeg, ones_k)

    featpad = jnp.concatenate(
        [feature, jnp.zeros((ACC_ROWS - N_NODES, D), jnp.float32)])
    out = _tc_apply(p, degp.reshape(NC, ACC_ROWS, 1), featpad, W,
                    b.reshape(1, D))
    return out[:N_NODES]


# speedup vs baseline: 1.5333x; 1.5333x over previous
import functools

import jax
import jax.numpy as jnp
from jax import lax
from jax.experimental import pallas as pl
from jax.experimental.pallas import tpu as pltpu
from jax.experimental.pallas import tpu_sc as plsc

N_NODES = 10000
N_EDGES = 320000
D = 128

NC = 2
NS = 16
NW = NC * NS

K = 128
CH = 79
EDGES_PER_TILE = K * CH
E_PAD = NW * EDGES_PER_TILE
ACC_ROWS = 10240
ROWS_PER_TILE = ACC_ROWS // NS


def _sc_body(feat_hbm, src_hbm, dst_hbm, zacc_hbm, zdeg_hbm, ones_hbm,
             p_hbm, degp_hbm,
             src_v, dst_v, rows_v, ones_v, acc_sh, deg_sh, sem, dsem):
    cid = lax.axis_index("c")
    sid = lax.axis_index("s")
    wid = cid * NS + sid

    r0 = sid * ROWS_PER_TILE
    pltpu.sync_copy(zacc_hbm.at[pl.ds(r0, ROWS_PER_TILE)],
                    acc_sh.at[pl.ds(r0, ROWS_PER_TILE)])
    pltpu.sync_copy(zdeg_hbm.at[pl.ds(r0, ROWS_PER_TILE)],
                    deg_sh.at[pl.ds(r0, ROWS_PER_TILE)])
    pltpu.sync_copy(src_hbm.at[wid], src_v)
    pltpu.sync_copy(dst_hbm.at[wid], dst_v)
    pltpu.sync_copy(ones_hbm, ones_v)
    plsc.subcore_barrier()

    def chunk(c, carry):
        pltpu.async_copy(feat_hbm.at[src_v.at[c]], rows_v, sem).wait()
        pltpu.sync_copy(rows_v, acc_sh.at[dst_v.at[c]], add=True)
        pltpu.async_copy(ones_v, deg_sh.at[dst_v.at[c]], dsem, add=True)
        return carry

    lax.fori_loop(0, CH, chunk, 0)
    pltpu.make_async_copy(dst_hbm.at[wid], dst_v, dsem).wait()
    plsc.subcore_barrier()

    pltpu.sync_copy(acc_sh.at[pl.ds(r0, ROWS_PER_TILE)],
                    p_hbm.at[cid, pl.ds(r0, ROWS_PER_TILE)])
    pltpu.sync_copy(deg_sh.at[pl.ds(r0, ROWS_PER_TILE)],
                    degp_hbm.at[pl.ds(cid * ACC_ROWS + r0, ROWS_PER_TILE)])


_sc_scatter = functools.partial(
    pl.kernel,
    out_type=(jax.ShapeDtypeStruct((NC, ACC_ROWS, D), jnp.float32),
              jax.ShapeDtypeStruct((NC * ACC_ROWS,), jnp.float32)),
    mesh=plsc.VectorSubcoreMesh(core_axis_name="c", subcore_axis_name="s",
                                num_cores=NC, num_subcores=NS),
    scratch_types=[
        pltpu.VMEM((CH, K), jnp.int32),
        pltpu.VMEM((CH, K), jnp.int32),
        pltpu.VMEM((K, D), jnp.float32),
        pltpu.VMEM((K,), jnp.float32),
        pltpu.VMEM_SHARED((ACC_ROWS, D), jnp.float32),
        pltpu.VMEM_SHARED((ACC_ROWS,), jnp.float32),
        pltpu.SemaphoreType.DMA,
        pltpu.SemaphoreType.DMA,
    ],
)(_sc_body)


def _tc_body(p_ref, deg_ref, feat_ref, w_ref, b_ref, out_ref):
    s = p_ref[0] + p_ref[1]
    d = deg_ref[0] + deg_ref[1]
    mean = s / jnp.maximum(d, 1.0)
    h = jnp.where(d > 0, mean, feat_ref[...])
    y = lax.dot_general(h, w_ref[...], (((1,), (1,)), ((), ())),
                        preferred_element_type=jnp.float32)
    out_ref[...] = jnp.maximum(y + b_ref[...], 0.0)


TC_R = 1280


def _tc_apply(p, degp, featpad, W, b2):
    return pl.pallas_call(
        _tc_body,
        grid=(ACC_ROWS // TC_R,),
        in_specs=[
            pl.BlockSpec((NC, TC_R, D), lambda i: (0, i, 0)),
            pl.BlockSpec((NC, TC_R, 1), lambda i: (0, i, 0)),
            pl.BlockSpec((TC_R, D), lambda i: (i, 0)),
            pl.BlockSpec((D, D), lambda i: (0, 0)),
            pl.BlockSpec((1, D), lambda i: (0, 0)),
        ],
        out_specs=pl.BlockSpec((TC_R, D), lambda i: (i, 0)),
        out_shape=jax.ShapeDtypeStruct((ACC_ROWS, D), jnp.float32),
    )(p, degp, featpad, W, b2)


def kernel(feature, edge_index, W, b):
    pad = E_PAD - N_EDGES
    src = jnp.concatenate([edge_index[0], jnp.zeros((pad,), jnp.int32)])
    dst = jnp.concatenate(
        [edge_index[1], jnp.full((pad,), N_NODES, jnp.int32)])
    src3 = src.reshape(NW, CH, K)
    dst3 = dst.reshape(NW, CH, K)
    zacc = jnp.zeros((ACC_ROWS, D), jnp.float32)
    zdeg = jnp.zeros((ACC_ROWS,), jnp.float32)
    ones_k = jnp.ones((K,), jnp.float32)

    p, degp = _sc_scatter(feature, src3, dst3, zacc, zdeg, ones_k)

    featpad = jnp.concatenate(
        [feature, jnp.zeros((ACC_ROWS - N_NODES, D), jnp.float32)])
    out = _tc_apply(p, degp.reshape(NC, ACC_ROWS, 1), featpad, W,
                    b.reshape(1, D))
    return out[:N_NODES]
